# baseline (speedup 1.0000x reference)
import functools

import jax
import jax.numpy as jnp
from jax.experimental import pallas as pl
from jax.experimental.pallas import tpu as pltpu

N = 10000
E = 320000
D = 128
B = 16384

_BLK = 1024


def _dense_body(ue_ref, hu_ref, ie_ref, hi_ref,
                wsu_ref, bsu_ref, wnu_ref, bnu_ref, wfu_ref,
                wsi_ref, bsi_ref, wni_ref, bni_ref, wfi_ref,
                out_ref):
    f32 = jnp.float32
    hp = "highest"
    us = jax.nn.relu(jnp.dot(ue_ref[...], wsu_ref[...],
                             preferred_element_type=f32, precision=hp)
                     + bsu_ref[...])
    un = jax.nn.relu(jnp.dot(hu_ref[...], wnu_ref[...],
                             preferred_element_type=f32, precision=hp)
                     + bnu_ref[...])
    uv = jax.nn.relu(jnp.dot(us, wfu_ref[0:D, :],
                             preferred_element_type=f32, precision=hp)
                     + jnp.dot(un, wfu_ref[D:2 * D, :],
                               preferred_element_type=f32, precision=hp))
    isf = jax.nn.relu(jnp.dot(ie_ref[...], wsi_ref[...],
                              preferred_element_type=f32, precision=hp)
                      + bsi_ref[...])
    inb = jax.nn.relu(jnp.dot(hi_ref[...], wni_ref[...],
                              preferred_element_type=f32, precision=hp)
                      + bni_ref[...])
    iv = jax.nn.relu(jnp.dot(isf, wfi_ref[0:D, :],
                             preferred_element_type=f32, precision=hp)
                     + jnp.dot(inb, wfi_ref[D:2 * D, :],
                               preferred_element_type=f32, precision=hp))
    out_ref[:, 0:D] = uv
    out_ref[:, D:2 * D] = iv


def _dense_stage(ue_rows, hu_rows, ie_rows, hi_rows,
                 W_self_u, b_self_u, W_nb_u, b_nb_u, W_fc_u,
                 W_self_i, b_self_i, W_nb_i, b_nb_i, W_fc_i):
    row_spec = pl.BlockSpec((_BLK, D), lambda b: (b, 0))
    wspec = lambda r, c: pl.BlockSpec((r, c), lambda b: (0, 0))
    bspec = pl.BlockSpec((D,), lambda b: (0,))
    return pl.pallas_call(
        _dense_body,
        grid=(B // _BLK,),
        in_specs=[row_spec, row_spec, row_spec, row_spec,
                  wspec(D, D), bspec, wspec(D, D), bspec, wspec(2 * D, D),
                  wspec(D, D), bspec, wspec(D, D), bspec, wspec(2 * D, D)],
        out_specs=pl.BlockSpec((_BLK, 2 * D), lambda b: (b, 0)),
        out_shape=jax.ShapeDtypeStruct((B, 2 * D), jnp.float32),
    )(ue_rows, hu_rows, ie_rows, hi_rows,
      W_self_u, b_self_u, W_nb_u, b_nb_u, W_fc_u,
      W_self_i, b_self_i, W_nb_i, b_nb_i, W_fc_i)


def _seg_softmax_agg(e, src_feat, dst, num_seg):
    m = jax.ops.segment_max(e, dst, num_segments=num_seg)
    ex = jnp.exp(e - m[dst])
    s = jax.ops.segment_sum(ex, dst, num_segments=num_seg)
    alpha = ex / s[dst]
    return jax.ops.segment_sum(alpha[:, None] * src_feat, dst,
                               num_segments=num_seg)


def kernel(user_emb, item_emb, W_attn_u, W_attn_i, W_fc_u, W_fc_i,
           W_self_u, b_self_u, W_self_i, b_self_i,
           W_nb_u, b_nb_u, W_nb_i, b_nb_i, edge_iu, edge_ui, u, i):
    src_u, dst_u = edge_iu[0], edge_iu[1]
    a_src_u = item_emb @ W_attn_u[:D, 0]
    a_dst_u = user_emb @ W_attn_u[D:, 0]
    e_u = jax.nn.leaky_relu(a_src_u[src_u] + a_dst_u[dst_u])
    h_user = _seg_softmax_agg(e_u, item_emb[src_u], dst_u, N)
    src_i, dst_i = edge_ui[0], edge_ui[1]
    a_src_i = user_emb @ W_attn_i[:D, 0]
    a_dst_i = item_emb @ W_attn_i[D:, 0]
    e_i = jax.nn.leaky_relu(a_src_i[src_i] + a_dst_i[dst_i])
    h_item = _seg_softmax_agg(e_i, user_emb[src_i], dst_i, N)

    return _dense_stage(user_emb[u], h_user[u], item_emb[i], h_item[i],
                        W_self_u, b_self_u, W_nb_u, b_nb_u, W_fc_u,
                        W_self_i, b_self_i, W_nb_i, b_nb_i, W_fc_i)


# speedup vs baseline: 14.9217x; 14.9217x over previous
import functools

import jax
import jax.numpy as jnp
from jax import lax
from jax.experimental import pallas as pl
from jax.experimental.pallas import tpu as pltpu
from jax.experimental.pallas import tpu_sc as plsc

N = 10000
E = 320000
D = 128
B = 16384

DP = 144
NT = 16
NC = 2
NW = NC * NT
EW = E // NW
EC = 80
NCHUNK = EW // EC
NPAD = 10240
RPT = NPAD // NT
BPT = B // NT
BPW = B // NW
GC = 64

_PREP_BLK = 2000
_DENSE_BLK = 1024


def _prep_body(ue_ref, ie_ref, wu2_ref, wi2_ref,
               up_ref, ip_ref, au_ref, ai_ref):
    hp = "highest"
    au = jnp.dot(ue_ref[...], wu2_ref[...],
                 preferred_element_type=jnp.float32, precision=hp)
    ai = jnp.dot(ie_ref[...], wi2_ref[...],
                 preferred_element_type=jnp.float32, precision=hp)
    tcol = jax.lax.broadcasted_iota(jnp.int32, (_PREP_BLK, DP - D), 1)
    up_ref[:, 0:D] = ue_ref[...]
    up_ref[:, D:DP] = jnp.where(tcol == 0, 1.0,
                                jnp.where(tcol == 1, au[:, 1:2], 0.0))
    ip_ref[:, 0:D] = ie_ref[...]
    ip_ref[:, D:DP] = jnp.where(tcol == 0, 1.0,
                                jnp.where(tcol == 1, ai[:, 0:1], 0.0))
    au_ref[...] = au
    ai_ref[...] = ai


def _prep(user_emb, item_emb, Wu2, Wi2):
    row = pl.BlockSpec((_PREP_BLK, D), lambda b: (b, 0))
    wsp = pl.BlockSpec((D, 2), lambda b: (0, 0))
    return pl.pallas_call(
        _prep_body,
        grid=(N // _PREP_BLK,),
        in_specs=[row, row, wsp, wsp],
        out_specs=[pl.BlockSpec((_PREP_BLK, DP), lambda b: (b, 0)),
                   pl.BlockSpec((_PREP_BLK, DP), lambda b: (b, 0)),
                   pl.BlockSpec((_PREP_BLK, 2), lambda b: (b, 0)),
                   pl.BlockSpec((_PREP_BLK, 2), lambda b: (b, 0))],
        out_shape=[jax.ShapeDtypeStruct((N, DP), jnp.float32),
                   jax.ShapeDtypeStruct((N, DP), jnp.float32),
                   jax.ShapeDtypeStruct((N, 2), jnp.float32),
                   jax.ShapeDtypeStruct((N, 2), jnp.float32)],
    )(user_emb, item_emb, Wu2, Wi2)


def _sc_edges_body(up_hbm, ip_hbm, adu_hbm, adi_hbm,
                   srcu_hbm, dstu_hbm, srci_hbm, dsti_hbm,
                   u_hbm, i_hbm, zr_hbm,
                   pu_out, pi_out, ur_out, ir_out,
                   acc, ad_v, sidx, didx, rows, pbuf, gbuf, bidx,
                   sem):
    cid = lax.axis_index("c")
    sid = lax.axis_index("s")
    wid = sid * NC + cid

    def one_side(ad_hbm, src_hbm, dst_hbm, table_hbm, bat_hbm, part_out):
        pltpu.sync_copy(ad_hbm, ad_v)
        pltpu.sync_copy(zr_hbm, acc.at[pl.ds(sid * RPT, RPT), :])
        plsc.subcore_barrier()

        def chunk(c, carry):
            base = wid * EW + c * EC
            pltpu.sync_copy(src_hbm.at[pl.ds(base, EC)], sidx)
            pltpu.sync_copy(dst_hbm.at[pl.ds(base, EC)], didx)
            pltpu.async_copy(table_hbm.at[sidx], rows, sem).wait()
            for g in range(EC // 16):
                dv = didx[pl.ds(g * 16, 16)]
                lanes = jax.lax.iota(jnp.int32, 16) + g * 16
                a_src = plsc.load_gather(rows, [lanes,
                                                jnp.zeros((16,), jnp.int32)
                                                + (D + 1)])
                a = a_src + plsc.load_gather(ad_v, [dv])
                e = jnp.where(a >= 0.0, a, a * 0.01)
                pbuf[pl.ds(g * 16, 16)] = jnp.exp(e)

            def scale(j, carry2):
                pj = plsc.load_gather(pbuf, [jnp.zeros((16,), jnp.int32) + j])
                for k in range(DP // 16):
                    rows[j, pl.ds(k * 16, 16)] = rows[j, pl.ds(k * 16, 16)] * pj
                return carry2

            lax.fori_loop(0, EC, scale, 0)
            pltpu.sync_copy(rows, acc.at[didx], add=True)
            return carry

        lax.fori_loop(0, NCHUNK, chunk, 0)
        plsc.subcore_barrier()

        def bat_part(c, carry):
            off = sid * BPT + c * GC
            pltpu.sync_copy(bat_hbm.at[pl.ds(off, GC)], bidx)
            pltpu.async_copy(acc.at[bidx], gbuf, sem).wait()
            pltpu.sync_copy(gbuf, part_out.at[cid, pl.ds(off, GC), :])
            return carry

        lax.fori_loop(0, BPT // GC, bat_part, 0)
        plsc.subcore_barrier()

    one_side(adu_hbm, srcu_hbm, dstu_hbm, ip_hbm, u_hbm, pu_out)
    one_side(adi_hbm, srci_hbm, dsti_hbm, up_hbm, i_hbm, pi_out)

    def bat_emb2(table_hbm, bat_hbm, emb_out):
        def go(c, carry):
            off = wid * BPW + c * GC
            pltpu.sync_copy(bat_hbm.at[pl.ds(off, GC)], bidx)
            pltpu.async_copy(table_hbm.at[bidx], gbuf, sem).wait()
            pltpu.sync_copy(gbuf, emb_out.at[pl.ds(off, GC), :])
            return carry
        lax.fori_loop(0, BPW // GC, go, 0)

    bat_emb2(up_hbm, u_hbm, ur_out)
    bat_emb2(ip_hbm, i_hbm, ir_out)


def _sc_edges(user_pad, item_pad, adu, adi,
              src_u, dst_u, src_i, dst_i, u_idx, i_idx, zrows):
    mesh = plsc.VectorSubcoreMesh(core_axis_name="c", subcore_axis_name="s")
    f32 = jnp.float32
    kern = functools.partial(
        pl.kernel,
        mesh=mesh,
        compiler_params=pltpu.CompilerParams(needs_layout_passes=False,
                                             use_tc_tiling_on_sc=False),
        out_type=(jax.ShapeDtypeStruct((NC, B, DP), f32),
                  jax.ShapeDtypeStruct((NC, B, DP), f32),
                  jax.ShapeDtypeStruct((B, DP), f32),
                  jax.ShapeDtypeStruct((B, DP), f32)),
        scratch_types=[
            pltpu.VMEM_SHARED((NPAD, DP), f32),
            pltpu.VMEM((N,), f32),
            pltpu.VMEM((EC,), jnp.int32),
            pltpu.VMEM((EC,), jnp.int32),
            pltpu.VMEM((EC, DP), f32),
            pltpu.VMEM((EC,), f32),
            pltpu.VMEM((GC, DP), f32),
            pltpu.VMEM((GC,), jnp.int32),
            pltpu.SemaphoreType.DMA,
        ],
    )(_sc_edges_body)
    return kern(user_pad, item_pad, adu, adi,
                src_u, dst_u, src_i, dst_i, u_idx, i_idx, zrows)


def _dense_body(pu_ref, pi_ref, ur_ref, ir_ref,
                wsu_ref, bsu_ref, wnu_ref, bnu_ref, wfu_ref,
                wsi_ref, bsi_ref, wni_ref, bni_ref, wfi_ref,
                out_ref):
    f32 = jnp.float32
    hp = "highest"

    def side(p_ref, e_ref, ws, bs, wn, bn, wf):
        num = p_ref[0, :, 0:D] + p_ref[1, :, 0:D]
        s = p_ref[0, :, D:D + 1] + p_ref[1, :, D:D + 1]
        s = jnp.where(s > 0.0, s, 1.0)
        h = num / s
        nb = jax.nn.relu(jnp.dot(h, wn[...], preferred_element_type=f32,
                                 precision=hp) + bn[...])
        sf = jax.nn.relu(jnp.dot(e_ref[:, 0:D], ws[...],
                                 preferred_element_type=f32,
                                 precision=hp) + bs[...])
        return jax.nn.relu(
            jnp.dot(sf, wf[0:D, :], preferred_element_type=f32, precision=hp)
            + jnp.dot(nb, wf[D:2 * D, :], preferred_element_type=f32,
                      precision=hp))

    out_ref[:, 0:D] = side(pu_ref, ur_ref, wsu_ref, bsu_ref,
                           wnu_ref, bnu_ref, wfu_ref)
    out_ref[:, D:2 * D] = side(pi_ref, ir_ref, wsi_ref, bsi_ref,
                               wni_ref, bni_ref, wfi_ref)


def _dense_stage(part_u, part_i, ue_rows, ie_rows,
                 W_self_u, b_self_u, W_nb_u, b_nb_u, W_fc_u,
                 W_self_i, b_self_i, W_nb_i, b_nb_i, W_fc_i):
    psp = pl.BlockSpec((NC, _DENSE_BLK, DP), lambda b: (0, b, 0))
    rsp = pl.BlockSpec((_DENSE_BLK, DP), lambda b: (b, 0))
    wsp = lambda r, c: pl.BlockSpec((r, c), lambda b: (0, 0))
    bsp = pl.BlockSpec((D,), lambda b: (0,))
    return pl.pallas_call(
        _dense_body,
        grid=(B // _DENSE_BLK,),
        in_specs=[psp, psp, rsp, rsp,
                  wsp(D, D), bsp, wsp(D, D), bsp, wsp(2 * D, D),
                  wsp(D, D), bsp, wsp(D, D), bsp, wsp(2 * D, D)],
        out_specs=pl.BlockSpec((_DENSE_BLK, 2 * D), lambda b: (b, 0)),
        out_shape=jax.ShapeDtypeStruct((B, 2 * D), jnp.float32),
    )(part_u, part_i, ue_rows, ie_rows,
      W_self_u, b_self_u, W_nb_u, b_nb_u, W_fc_u,
      W_self_i, b_self_i, W_nb_i, b_nb_i, W_fc_i)


def kernel(user_emb, item_emb, W_attn_u, W_attn_i, W_fc_u, W_fc_i,
           W_self_u, b_self_u, W_self_i, b_self_i,
           W_nb_u, b_nb_u, W_nb_i, b_nb_i, edge_iu, edge_ui, u, i):
    Wu2 = jnp.concatenate([W_attn_u[D:], W_attn_i[:D]], axis=1)
    Wi2 = jnp.concatenate([W_attn_u[:D], W_attn_i[D:]], axis=1)
    user_pad, item_pad, AU, AI = _prep(user_emb, item_emb, Wu2, Wi2)
    adu = AU[:, 0]
    adi = AI[:, 1]
    zrows = jnp.zeros((RPT, DP), jnp.float32)
    part_u, part_i, ue_rows, ie_rows = _sc_edges(
        user_pad, item_pad, adu, adi,
        edge_iu[0], edge_iu[1], edge_ui[0], edge_ui[1], u, i, zrows)
    return _dense_stage(part_u, part_i, ue_rows, ie_rows,
                        W_self_u, b_self_u, W_nb_u, b_nb_u, W_fc_u,
                        W_self_i, b_self_i, W_nb_i, b_nb_i, W_fc_i)


# speedup vs baseline: 23.3296x; 1.5635x over previous
import functools

import jax
import jax.numpy as jnp
from jax import lax
from jax.experimental import pallas as pl
from jax.experimental.pallas import tpu as pltpu
from jax.experimental.pallas import tpu_sc as plsc

N = 10000
E = 320000
D = 128
B = 16384

DP = 144
NT = 16
NC = 2
NW = NC * NT
EW = E // NT
EC = 80
NCHUNK = EW // EC
NPAD = 10240
RPT = NPAD // NT
BPT = B // NT
BPW = B // NW
GC = 64

_PREP_BLK = 2000
_DENSE_BLK = 1024


def _prep_body(ue_ref, ie_ref, wu2_ref, wi2_ref,
               up_ref, ip_ref, au_ref, ai_ref):
    hp = "highest"
    au = jnp.dot(ue_ref[...], wu2_ref[...],
                 preferred_element_type=jnp.float32, precision=hp)
    ai = jnp.dot(ie_ref[...], wi2_ref[...],
                 preferred_element_type=jnp.float32, precision=hp)
    tcol = jax.lax.broadcasted_iota(jnp.int32, (_PREP_BLK, DP - D), 1)
    up_ref[:, 0:D] = ue_ref[...]
    up_ref[:, D:DP] = jnp.where(tcol == 0, 1.0,
                                jnp.where(tcol == 1, au[:, 1:2], 0.0))
    ip_ref[:, 0:D] = ie_ref[...]
    ip_ref[:, D:DP] = jnp.where(tcol == 0, 1.0,
                                jnp.where(tcol == 1, ai[:, 0:1], 0.0))
    acol = jax.lax.broadcasted_iota(jnp.int32, (_PREP_BLK, 8), 1)
    au_ref[...] = jnp.where(acol == 0, au[:, 0:1], 0.0)
    ai_ref[...] = jnp.where(acol == 0, ai[:, 1:2], 0.0)


def _prep(user_emb, item_emb, Wu2, Wi2):
    row = pl.BlockSpec((_PREP_BLK, D), lambda b: (b, 0))
    wsp = pl.BlockSpec((D, 2), lambda b: (0, 0))
    return pl.pallas_call(
        _prep_body,
        grid=(N // _PREP_BLK,),
        in_specs=[row, row, wsp, wsp],
        out_specs=[pl.BlockSpec((_PREP_BLK, DP), lambda b: (b, 0)),
                   pl.BlockSpec((_PREP_BLK, DP), lambda b: (b, 0)),
                   pl.BlockSpec((_PREP_BLK, 8), lambda b: (b, 0)),
                   pl.BlockSpec((_PREP_BLK, 8), lambda b: (b, 0))],
        out_shape=[jax.ShapeDtypeStruct((N, DP), jnp.float32),
                   jax.ShapeDtypeStruct((N, DP), jnp.float32),
                   jax.ShapeDtypeStruct((N, 8), jnp.float32),
                   jax.ShapeDtypeStruct((N, 8), jnp.float32)],
    )(user_emb, item_emb, Wu2, Wi2)


def _sc_edges_body(up_hbm, ip_hbm, adu_hbm, adi_hbm,
                   srcu_hbm, dstu_hbm, srci_hbm, dsti_hbm,
                   u_hbm, i_hbm, zr_hbm,
                   pu_out, pi_out, ur_out, ir_out,
                   acc, sidx0, didx0, rows0, adb0, sidx1, didx1, rows1, adb1,
                   pbuf, gbuf, bidx, semr0, sema0, semr1, sema1, sem):
    cid = lax.axis_index("c")
    sid = lax.axis_index("s")
    bufs = ((sidx0, didx0, rows0, adb0, semr0, sema0),
            (sidx1, didx1, rows1, adb1, semr1, sema1))

    def one_side(ad_hbm, src_hbm, dst_hbm, table_hbm, dst_tab_hbm,
                 bat_hbm, part_out, emb_out):
        pltpu.sync_copy(zr_hbm, acc.at[pl.ds(sid * RPT, RPT), :])
        plsc.subcore_barrier()

        def fetch(c, buf):
            si, di, rw, ab, sr, sa = buf
            base = sid * EW + c * EC
            pltpu.sync_copy(src_hbm.at[pl.ds(base, EC)], si)
            pltpu.sync_copy(dst_hbm.at[pl.ds(base, EC)], di)
            pltpu.async_copy(table_hbm.at[si], rw, sr)
            pltpu.async_copy(ad_hbm.at[di], ab, sa)

        def process(buf):
            si, di, rw, ab, sr, sa = buf
            pltpu.make_async_copy(table_hbm.at[si], rw, sr).wait()
            pltpu.make_async_copy(ad_hbm.at[di], ab, sa).wait()
            z16 = jnp.zeros((16,), jnp.int32)
            for g in range(EC // 16):
                lanes = jax.lax.iota(jnp.int32, 16) + g * 16
                a_src = plsc.load_gather(rw, [lanes, z16 + (D + 1)])
                a = a_src + ab[pl.ds(g * 16, 16)]
                e = jnp.where(a >= 0.0, a, a * 0.01)
                pbuf[pl.ds(g * 16, 16)] = jnp.exp(e)

            @plsc.parallel_loop(0, EC, 1, unroll=2)
            def _(j):
                pj = plsc.load_gather(pbuf, [jnp.zeros((16,), jnp.int32) + j])
                for k in range(DP // 16):
                    rw[j, pl.ds(k * 16, 16)] = rw[j, pl.ds(k * 16, 16)] * pj

            pltpu.sync_copy(rw, acc.at[di], add=True)

        fetch(0, bufs[0])

        def pair(t, carry):
            c0 = 2 * t
            fetch(c0 + 1, bufs[1])
            process(bufs[0])
            fetch(c0 + 2, bufs[0])
            process(bufs[1])
            return carry

        lax.fori_loop(0, (NCHUNK - 1) // 2, pair, 0)
        if NCHUNK % 2:
            process(bufs[0])
        else:
            fetch(NCHUNK - 1, bufs[1])
            process(bufs[0])
            process(bufs[1])
        plsc.subcore_barrier()

        def bat(c, carry):
            off = sid * BPT + c * GC
            pltpu.sync_copy(bat_hbm.at[pl.ds(off, GC)], bidx)
            pltpu.async_copy(acc.at[bidx], gbuf, sem).wait()
            pltpu.sync_copy(gbuf, part_out.at[pl.ds(off, GC), :])
            pltpu.async_copy(dst_tab_hbm.at[bidx], gbuf, sem).wait()
            pltpu.sync_copy(gbuf, emb_out.at[pl.ds(off, GC), :])
            return carry

        lax.fori_loop(0, BPT // GC, bat, 0)

    @pl.when(cid == 0)
    def _():
        one_side(adu_hbm, srcu_hbm, dstu_hbm, ip_hbm, up_hbm,
                 u_hbm, pu_out, ur_out)

    @pl.when(cid == 1)
    def _():
        one_side(adi_hbm, srci_hbm, dsti_hbm, up_hbm, ip_hbm,
                 i_hbm, pi_out, ir_out)


def _sc_edges(user_pad, item_pad, adu, adi,
              src_u, dst_u, src_i, dst_i, u_idx, i_idx, zrows):
    mesh = plsc.VectorSubcoreMesh(core_axis_name="c", subcore_axis_name="s")
    f32 = jnp.float32
    kern = functools.partial(
        pl.kernel,
        mesh=mesh,
        compiler_params=pltpu.CompilerParams(needs_layout_passes=False,
                                             use_tc_tiling_on_sc=False),
        out_type=(jax.ShapeDtypeStruct((B, DP), f32),
                  jax.ShapeDtypeStruct((B, DP), f32),
                  jax.ShapeDtypeStruct((B, DP), f32),
                  jax.ShapeDtypeStruct((B, DP), f32)),
        scratch_types=[
            pltpu.VMEM_SHARED((NPAD, DP), f32),
            pltpu.VMEM((EC,), jnp.int32),
            pltpu.VMEM((EC,), jnp.int32),
            pltpu.VMEM((EC, DP), f32),
            pltpu.VMEM((EC,), f32),
            pltpu.VMEM((EC,), jnp.int32),
            pltpu.VMEM((EC,), jnp.int32),
            pltpu.VMEM((EC, DP), f32),
            pltpu.VMEM((EC,), f32),
            pltpu.VMEM((EC,), f32),
            pltpu.VMEM((GC, DP), f32),
            pltpu.VMEM((GC,), jnp.int32),
            pltpu.SemaphoreType.DMA,
            pltpu.SemaphoreType.DMA,
            pltpu.SemaphoreType.DMA,
            pltpu.SemaphoreType.DMA,
            pltpu.SemaphoreType.DMA,
        ],
    )(_sc_edges_body)
    return kern(user_pad, item_pad, adu, adi,
                src_u, dst_u, src_i, dst_i, u_idx, i_idx, zrows)


def _dense_body(pu_ref, pi_ref, ur_ref, ir_ref,
                wsu_ref, bsu_ref, wnu_ref, bnu_ref, wfu_ref,
                wsi_ref, bsi_ref, wni_ref, bni_ref, wfi_ref,
                out_ref):
    f32 = jnp.float32
    hp = "highest"

    def side(p_ref, e_ref, ws, bs, wn, bn, wf):
        num = p_ref[:, 0:D]
        s = p_ref[:, D:D + 1]
        s = jnp.where(s > 0.0, s, 1.0)
        h = num / s
        nb = jax.nn.relu(jnp.dot(h, wn[...], preferred_element_type=f32,
                                 precision=hp) + bn[...])
        sf = jax.nn.relu(jnp.dot(e_ref[:, 0:D], ws[...],
                                 preferred_element_type=f32,
                                 precision=hp) + bs[...])
        return jax.nn.relu(
            jnp.dot(sf, wf[0:D, :], preferred_element_type=f32, precision=hp)
            + jnp.dot(nb, wf[D:2 * D, :], preferred_element_type=f32,
                      precision=hp))

    out_ref[:, 0:D] = side(pu_ref, ur_ref, wsu_ref, bsu_ref,
                           wnu_ref, bnu_ref, wfu_ref)
    out_ref[:, D:2 * D] = side(pi_ref, ir_ref, wsi_ref, bsi_ref,
                               wni_ref, bni_ref, wfi_ref)


def _dense_stage(part_u, part_i, ue_rows, ie_rows,
                 W_self_u, b_self_u, W_nb_u, b_nb_u, W_fc_u,
                 W_self_i, b_self_i, W_nb_i, b_nb_i, W_fc_i):
    psp = pl.BlockSpec((_DENSE_BLK, DP), lambda b: (b, 0))
    rsp = pl.BlockSpec((_DENSE_BLK, DP), lambda b: (b, 0))
    wsp = lambda r, c: pl.BlockSpec((r, c), lambda b: (0, 0))
    bsp = pl.BlockSpec((D,), lambda b: (0,))
    return pl.pallas_call(
        _dense_body,
        grid=(B // _DENSE_BLK,),
        in_specs=[psp, psp, rsp, rsp,
                  wsp(D, D), bsp, wsp(D, D), bsp, wsp(2 * D, D),
                  wsp(D, D), bsp, wsp(D, D), bsp, wsp(2 * D, D)],
        out_specs=pl.BlockSpec((_DENSE_BLK, 2 * D), lambda b: (b, 0)),
        out_shape=jax.ShapeDtypeStruct((B, 2 * D), jnp.float32),
    )(part_u, part_i, ue_rows, ie_rows,
      W_self_u, b_self_u, W_nb_u, b_nb_u, W_fc_u,
      W_self_i, b_self_i, W_nb_i, b_nb_i, W_fc_i)


def kernel(user_emb, item_emb, W_attn_u, W_attn_i, W_fc_u, W_fc_i,
           W_self_u, b_self_u, W_self_i, b_self_i,
           W_nb_u, b_nb_u, W_nb_i, b_nb_i, edge_iu, edge_ui, u, i):
    Wu2 = jnp.concatenate([W_attn_u[D:], W_attn_i[:D]], axis=1)
    Wi2 = jnp.concatenate([W_attn_u[:D], W_attn_i[D:]], axis=1)
    user_pad, item_pad, AU, AI = _prep(user_emb, item_emb, Wu2, Wi2)
    adu = AU[:, 0]
    adi = AI[:, 0]
    zrows = jnp.zeros((RPT, DP), jnp.float32)
    part_u, part_i, ue_rows, ie_rows = _sc_edges(
        user_pad, item_pad, adu, adi,
        edge_iu[0], edge_iu[1], edge_ui[0], edge_ui[1], u, i, zrows)
    return _dense_stage(part_u, part_i, ue_rows, ie_rows,
                        W_self_u, b_self_u, W_nb_u, b_nb_u, W_fc_u,
                        W_self_i, b_self_i, W_nb_i, b_nb_i, W_fc_i)


# speedup vs baseline: 30.5705x; 1.3104x over previous
import functools

import jax
import jax.numpy as jnp
from jax import lax
from jax.experimental import pallas as pl
from jax.experimental.pallas import tpu as pltpu
from jax.experimental.pallas import tpu_sc as plsc

N = 10000
E = 320000
D = 128
B = 16384

DP = 144
NT = 16
NC = 2
NW = NC * NT
EW = E // NT
EC = 80
NCHUNK = EW // EC
NPAD = 10240
RPT = NPAD // NT
BPT = B // NT
GC = 32
NP = NCHUNK // 2

_PREP_BLK = 2000
_DENSE_BLK = 1024


def _prep_body(ue_ref, ie_ref, wu2_ref, wi2_ref,
               up_ref, ip_ref, au_ref, ai_ref):
    hp = "highest"
    au = jnp.dot(ue_ref[...], wu2_ref[...],
                 preferred_element_type=jnp.float32, precision=hp)
    ai = jnp.dot(ie_ref[...], wi2_ref[...],
                 preferred_element_type=jnp.float32, precision=hp)
    tcol = jax.lax.broadcasted_iota(jnp.int32, (_PREP_BLK, DP - D), 1)
    up_ref[:, 0:D] = ue_ref[...]
    up_ref[:, D:DP] = jnp.where(tcol == 0, 1.0,
                                jnp.where(tcol == 1, au[:, 1:2], 0.0))
    ip_ref[:, 0:D] = ie_ref[...]
    ip_ref[:, D:DP] = jnp.where(tcol == 0, 1.0,
                                jnp.where(tcol == 1, ai[:, 0:1], 0.0))
    acol = jax.lax.broadcasted_iota(jnp.int32, (_PREP_BLK, 8), 1)
    au_ref[...] = jnp.where(acol == 0, au[:, 0:1], 0.0)
    ai_ref[...] = jnp.where(acol == 0, ai[:, 1:2], 0.0)


def _prep(user_emb, item_emb, Wu2, Wi2):
    row = pl.BlockSpec((_PREP_BLK, D), lambda b: (b, 0))
    wsp = pl.BlockSpec((D, 2), lambda b: (0, 0))
    return pl.pallas_call(
        _prep_body,
        grid=(N // _PREP_BLK,),
        in_specs=[row, row, wsp, wsp],
        out_specs=[pl.BlockSpec((_PREP_BLK, DP), lambda b: (b, 0)),
                   pl.BlockSpec((_PREP_BLK, DP), lambda b: (b, 0)),
                   pl.BlockSpec((_PREP_BLK, 8), lambda b: (b, 0)),
                   pl.BlockSpec((_PREP_BLK, 8), lambda b: (b, 0))],
        out_shape=[jax.ShapeDtypeStruct((N, DP), jnp.float32),
                   jax.ShapeDtypeStruct((N, DP), jnp.float32),
                   jax.ShapeDtypeStruct((N, 8), jnp.float32),
                   jax.ShapeDtypeStruct((N, 8), jnp.float32)],
    )(user_emb, item_emb, Wu2, Wi2)


def _sc_edges_body(up_hbm, ip_hbm, adu_hbm, adi_hbm,
                   srcu_hbm, dstu_hbm, srci_hbm, dsti_hbm,
                   u_hbm, i_hbm, zr_hbm,
                   pu_out, pi_out, ur_out, ir_out,
                   acc, is0, id0, rows0, adb0, is1, id1, rows1, adb1,
                   pbuf, g0, g1, bidx,
                   smi0, smi1, smr0, sma0, smr1, sma1):
    cid = lax.axis_index("c")
    sid = lax.axis_index("s")

    def one_side(ad_hbm, src2_hbm, dst2_hbm, table_hbm, dst_tab_hbm,
                 bat_hbm, part_out, emb_out):
        pltpu.sync_copy(zr_hbm, acc.at[pl.ds(sid * RPT, RPT), :])
        plsc.subcore_barrier()

        base = sid * NCHUNK

        def idx_row(p):
            return jnp.minimum(base + 2 * p, base + NCHUNK - 2)

        def idx_fetch(p, js, jd, jsem):
            r = idx_row(p)
            pltpu.async_copy(src2_hbm.at[pl.ds(r, 2), :], js, jsem)
            pltpu.async_copy(dst2_hbm.at[pl.ds(r, 2), :], jd, jsem)

        def idx_wait(p, js, jd, jsem):
            r = idx_row(p)
            pltpu.make_async_copy(src2_hbm.at[pl.ds(r, 2), :], js,
                                  jsem).wait()
            pltpu.make_async_copy(dst2_hbm.at[pl.ds(r, 2), :], jd,
                                  jsem).wait()

        def gather(js, jd, j, rw, ab, sr, sa):
            pltpu.async_copy(table_hbm.at[js.at[j]], rw, sr)
            pltpu.async_copy(ad_hbm.at[jd.at[j]], ab, sa)

        def gwait(js, jd, j, rw, ab, sr, sa):
            pltpu.make_async_copy(table_hbm.at[js.at[j]], rw, sr).wait()
            pltpu.make_async_copy(ad_hbm.at[jd.at[j]], ab, sa).wait()

        def compute_scatter(jd, j, rw, ab):
            z16 = jnp.zeros((16,), jnp.int32)
            for g in range(EC // 16):
                lanes = jax.lax.iota(jnp.int32, 16) + g * 16
                a_src = plsc.load_gather(rw, [lanes, z16 + (D + 1)])
                a = a_src + ab[pl.ds(g * 16, 16)]
                e = jnp.where(a >= 0.0, a, a * 0.01)
                pbuf[pl.ds(g * 16, 16)] = jnp.exp(e)

            @plsc.parallel_loop(0, EC, 1, unroll=2)
            def _(jj):
                pj = plsc.load_gather(pbuf,
                                      [jnp.zeros((16,), jnp.int32) + jj])
                for k in range(DP // 16):
                    rw[jj, pl.ds(k * 16, 16)] = (rw[jj, pl.ds(k * 16, 16)]
                                                 * pj)

            pltpu.sync_copy(rw, acc.at[jd.at[j]], add=True)

        idx_fetch(0, is0, id0, smi0)
        idx_wait(0, is0, id0, smi0)
        gather(is0, id0, 0, rows0, adb0, smr0, sma0)
        gather(is0, id0, 1, rows1, adb1, smr1, sma1)
        idx_fetch(1, is1, id1, smi1)

        def quad(u, carry):
            gwait(is0, id0, 0, rows0, adb0, smr0, sma0)
            compute_scatter(id0, 0, rows0, adb0)
            idx_wait(2 * u + 1, is1, id1, smi1)
            gather(is1, id1, 0, rows0, adb0, smr0, sma0)
            gwait(is0, id0, 1, rows1, adb1, smr1, sma1)
            compute_scatter(id0, 1, rows1, adb1)
            gather(is1, id1, 1, rows1, adb1, smr1, sma1)
            idx_fetch(2 * u + 2, is0, id0, smi0)
            gwait(is1, id1, 0, rows0, adb0, smr0, sma0)
            compute_scatter(id1, 0, rows0, adb0)
            idx_wait(2 * u + 2, is0, id0, smi0)
            gather(is0, id0, 0, rows0, adb0, smr0, sma0)
            gwait(is1, id1, 1, rows1, adb1, smr1, sma1)
            compute_scatter(id1, 1, rows1, adb1)
            gather(is0, id0, 1, rows1, adb1, smr1, sma1)
            idx_fetch(2 * u + 3, is1, id1, smi1)
            return carry

        lax.fori_loop(0, NP // 2, quad, 0)
        gwait(is0, id0, 0, rows0, adb0, smr0, sma0)
        compute_scatter(id0, 0, rows0, adb0)
        gwait(is0, id0, 1, rows1, adb1, smr1, sma1)
        compute_scatter(id0, 1, rows1, adb1)
        idx_wait(NP, is1, id1, smi1)
        plsc.subcore_barrier()

        def bat(c, carry):
            off = sid * BPT + c * GC
            pltpu.sync_copy(bat_hbm.at[pl.ds(off, GC)], bidx)
            pltpu.async_copy(acc.at[bidx], g0, smr0)
            pltpu.async_copy(dst_tab_hbm.at[bidx], g1, smr1)
            pltpu.make_async_copy(acc.at[bidx], g0, smr0).wait()
            pltpu.sync_copy(g0, part_out.at[pl.ds(off, GC), :])
            pltpu.make_async_copy(dst_tab_hbm.at[bidx], g1, smr1).wait()
            pltpu.sync_copy(g1, emb_out.at[pl.ds(off, GC), :])
            return carry

        lax.fori_loop(0, BPT // GC, bat, 0)

    @pl.when(cid == 0)
    def _():
        one_side(adu_hbm, srcu_hbm, dstu_hbm, ip_hbm, up_hbm,
                 u_hbm, pu_out, ur_out)

    @pl.when(cid == 1)
    def _():
        one_side(adi_hbm, srci_hbm, dsti_hbm, up_hbm, ip_hbm,
                 i_hbm, pi_out, ir_out)


def _sc_edges(user_pad, item_pad, adu, adi,
              src_u, dst_u, src_i, dst_i, u_idx, i_idx, zrows):
    mesh = plsc.VectorSubcoreMesh(core_axis_name="c", subcore_axis_name="s")
    f32 = jnp.float32
    kern = functools.partial(
        pl.kernel,
        mesh=mesh,
        compiler_params=pltpu.CompilerParams(needs_layout_passes=False,
                                             use_tc_tiling_on_sc=False),
        out_type=(jax.ShapeDtypeStruct((B, DP), f32),
                  jax.ShapeDtypeStruct((B, DP), f32),
                  jax.ShapeDtypeStruct((B, DP), f32),
                  jax.ShapeDtypeStruct((B, DP), f32)),
        scratch_types=[
            pltpu.VMEM_SHARED((NPAD, DP), f32),
            pltpu.VMEM((2, EC), jnp.int32),
            pltpu.VMEM((2, EC), jnp.int32),
            pltpu.VMEM((EC, DP), f32),
            pltpu.VMEM((EC,), f32),
            pltpu.VMEM((2, EC), jnp.int32),
            pltpu.VMEM((2, EC), jnp.int32),
            pltpu.VMEM((EC, DP), f32),
            pltpu.VMEM((EC,), f32),
            pltpu.VMEM((EC,), f32),
            pltpu.VMEM((GC, DP), f32),
            pltpu.VMEM((GC, DP), f32),
            pltpu.VMEM((GC,), jnp.int32),
            pltpu.SemaphoreType.DMA,
            pltpu.SemaphoreType.DMA,
            pltpu.SemaphoreType.DMA,
            pltpu.SemaphoreType.DMA,
            pltpu.SemaphoreType.DMA,
            pltpu.SemaphoreType.DMA,
        ],
    )(_sc_edges_body)
    return kern(user_pad, item_pad, adu, adi,
                src_u, dst_u, src_i, dst_i, u_idx, i_idx, zrows)


def _dense_body(pu_ref, pi_ref, ur_ref, ir_ref,
                wsu_ref, bsu_ref, wnu_ref, bnu_ref, wfu_ref,
                wsi_ref, bsi_ref, wni_ref, bni_ref, wfi_ref,
                out_ref):
    f32 = jnp.float32
    hp = "highest"

    def side(p_ref, e_ref, ws, bs, wn, bn, wf):
        num = p_ref[:, 0:D]
        s = p_ref[:, D:D + 1]
        s = jnp.where(s > 0.0, s, 1.0)
        h = num / s
        nb = jax.nn.relu(jnp.dot(h, wn[...], preferred_element_type=f32,
                                 precision=hp) + bn[...])
        sf = jax.nn.relu(jnp.dot(e_ref[:, 0:D], ws[...],
                                 preferred_element_type=f32,
                                 precision=hp) + bs[...])
        return jax.nn.relu(
            jnp.dot(sf, wf[0:D, :], preferred_element_type=f32, precision=hp)
            + jnp.dot(nb, wf[D:2 * D, :], preferred_element_type=f32,
                      precision=hp))

    out_ref[:, 0:D] = side(pu_ref, ur_ref, wsu_ref, bsu_ref,
                           wnu_ref, bnu_ref, wfu_ref)
    out_ref[:, D:2 * D] = side(pi_ref, ir_ref, wsi_ref, bsi_ref,
                               wni_ref, bni_ref, wfi_ref)


def _dense_stage(part_u, part_i, ue_rows, ie_rows,
                 W_self_u, b_self_u, W_nb_u, b_nb_u, W_fc_u,
                 W_self_i, b_self_i, W_nb_i, b_nb_i, W_fc_i):
    psp = pl.BlockSpec((_DENSE_BLK, DP), lambda b: (b, 0))
    rsp = pl.BlockSpec((_DENSE_BLK, DP), lambda b: (b, 0))
    wsp = lambda r, c: pl.BlockSpec((r, c), lambda b: (0, 0))
    bsp = pl.BlockSpec((D,), lambda b: (0,))
    return pl.pallas_call(
        _dense_body,
        grid=(B // _DENSE_BLK,),
        in_specs=[psp, psp, rsp, rsp,
                  wsp(D, D), bsp, wsp(D, D), bsp, wsp(2 * D, D),
                  wsp(D, D), bsp, wsp(D, D), bsp, wsp(2 * D, D)],
        out_specs=pl.BlockSpec((_DENSE_BLK, 2 * D), lambda b: (b, 0)),
        out_shape=jax.ShapeDtypeStruct((B, 2 * D), jnp.float32),
    )(part_u, part_i, ue_rows, ie_rows,
      W_self_u, b_self_u, W_nb_u, b_nb_u, W_fc_u,
      W_self_i, b_self_i, W_nb_i, b_nb_i, W_fc_i)


def kernel(user_emb, item_emb, W_attn_u, W_attn_i, W_fc_u, W_fc_i,
           W_self_u, b_self_u, W_self_i, b_self_i,
           W_nb_u, b_nb_u, W_nb_i, b_nb_i, edge_iu, edge_ui, u, i):
    Wu2 = jnp.concatenate([W_attn_u[D:], W_attn_i[:D]], axis=1)
    Wi2 = jnp.concatenate([W_attn_u[:D], W_attn_i[D:]], axis=1)
    user_pad, item_pad, AU, AI = _prep(user_emb, item_emb, Wu2, Wi2)
    adu = AU[:, 0]
    adi = AI[:, 0]
    zrows = jnp.zeros((RPT, DP), jnp.float32)
    part_u, part_i, ue_rows, ie_rows = _sc_edges(
        user_pad, item_pad, adu, adi,
        edge_iu[0].reshape(E // EC, EC), edge_iu[1].reshape(E // EC, EC),
        edge_ui[0].reshape(E // EC, EC), edge_ui[1].reshape(E // EC, EC),
        u, i, zrows)
    return _dense_stage(part_u, part_i, ue_rows, ie_rows,
                        W_self_u, b_self_u, W_nb_u, b_nb_u, W_fc_u,
                        W_self_i, b_self_i, W_nb_i, b_nb_i, W_fc_i)


# speedup vs baseline: 33.0275x; 1.0804x over previous
import functools

import jax
import jax.numpy as jnp
from jax import lax
from jax.experimental import pallas as pl
from jax.experimental.pallas import tpu as pltpu
from jax.experimental.pallas import tpu_sc as plsc

N = 10000
E = 320000
D = 128
B = 16384

DP = 144
NT = 16
NC = 2
NW = NC * NT
EW = E // NT
EC = 80
NCHUNK = EW // EC
NPAD = 10240
RPT = NPAD // NT
BPT = B // NT
GC = 32
NP = NCHUNK // 2

_PREP_BLK = 2000
_DENSE_BLK = 1024


def _prep_body(ue_ref, ie_ref, wu2_ref, wi2_ref,
               up_ref, ip_ref, au_ref, ai_ref):
    hp = "highest"
    au = jnp.dot(ue_ref[...], wu2_ref[...],
                 preferred_element_type=jnp.float32, precision=hp)
    ai = jnp.dot(ie_ref[...], wi2_ref[...],
                 preferred_element_type=jnp.float32, precision=hp)
    tcol = jax.lax.broadcasted_iota(jnp.int32, (_PREP_BLK, DP - D), 1)
    up_ref[:, 0:D] = ue_ref[...]
    up_ref[:, D:DP] = jnp.where(tcol == 0, 1.0,
                                jnp.where(tcol == 1, au[:, 1:2], 0.0))
    ip_ref[:, 0:D] = ie_ref[...]
    ip_ref[:, D:DP] = jnp.where(tcol == 0, 1.0,
                                jnp.where(tcol == 1, ai[:, 0:1], 0.0))
    acol = jax.lax.broadcasted_iota(jnp.int32, (_PREP_BLK, 8), 1)
    au_ref[...] = jnp.where(acol == 0, au[:, 0:1], 0.0)
    ai_ref[...] = jnp.where(acol == 0, ai[:, 1:2], 0.0)


def _prep(user_emb, item_emb, Wu2, Wi2):
    row = pl.BlockSpec((_PREP_BLK, D), lambda b: (b, 0))
    wsp = pl.BlockSpec((D, 2), lambda b: (0, 0))
    return pl.pallas_call(
        _prep_body,
        grid=(N // _PREP_BLK,),
        in_specs=[row, row, wsp, wsp],
        out_specs=[pl.BlockSpec((_PREP_BLK, DP), lambda b: (b, 0)),
                   pl.BlockSpec((_PREP_BLK, DP), lambda b: (b, 0)),
                   pl.BlockSpec((_PREP_BLK, 8), lambda b: (b, 0)),
                   pl.BlockSpec((_PREP_BLK, 8), lambda b: (b, 0))],
        out_shape=[jax.ShapeDtypeStruct((N, DP), jnp.float32),
                   jax.ShapeDtypeStruct((N, DP), jnp.float32),
                   jax.ShapeDtypeStruct((N, 8), jnp.float32),
                   jax.ShapeDtypeStruct((N, 8), jnp.float32)],
    )(user_emb, item_emb, Wu2, Wi2)


def _sc_edges_body(up_hbm, ip_hbm, adu_hbm, adi_hbm,
                   srcu_hbm, dstu_hbm, srci_hbm, dsti_hbm,
                   u_hbm, i_hbm, zr_hbm,
                   pu_out, pi_out, ur_out, ir_out,
                   acc, is0, id0, rows0, adb0, is1, id1, rows1, adb1,
                   pbuf, g0, g1, bidx,
                   smi0, smi1, smr0, sma0, smr1, sma1):
    cid = lax.axis_index("c")
    sid = lax.axis_index("s")

    def one_side(ad_hbm, src2_hbm, dst2_hbm, table_hbm, dst_tab_hbm,
                 bat_hbm, part_out, emb_out):
        pltpu.sync_copy(zr_hbm, acc.at[pl.ds(sid * RPT, RPT), :])
        plsc.subcore_barrier()

        base = sid * NCHUNK

        def idx_row(p):
            return jnp.minimum(base + 2 * p, base + NCHUNK - 2)

        def idx_fetch(p, js, jd, jsem):
            r = idx_row(p)
            pltpu.async_copy(src2_hbm.at[pl.ds(r, 2), :], js, jsem)
            pltpu.async_copy(dst2_hbm.at[pl.ds(r, 2), :], jd, jsem)

        def idx_wait(p, js, jd, jsem):
            r = idx_row(p)
            pltpu.make_async_copy(src2_hbm.at[pl.ds(r, 2), :], js,
                                  jsem).wait()
            pltpu.make_async_copy(dst2_hbm.at[pl.ds(r, 2), :], jd,
                                  jsem).wait()

        def gather(js, jd, j, rw, ab, sr, sa):
            pltpu.async_copy(table_hbm.at[js.at[j]], rw, sr)
            pltpu.async_copy(ad_hbm.at[jd.at[j]], ab, sa)

        def gwait(js, jd, j, rw, ab, sr, sa):
            pltpu.make_async_copy(table_hbm.at[js.at[j]], rw, sr).wait()
            pltpu.make_async_copy(ad_hbm.at[jd.at[j]], ab, sa).wait()

        def compute_scatter(jd, j, rw, ab):
            z16 = jnp.zeros((16,), jnp.int32)
            for g in range(EC // 16):
                lanes = jax.lax.iota(jnp.int32, 16) + g * 16
                a_src = plsc.load_gather(rw, [lanes, z16 + (D + 1)])
                a = a_src + ab[pl.ds(g * 16, 16)]
                e = jnp.where(a >= 0.0, a, a * 0.01)
                pbuf[pl.ds(g * 16, 16)] = jnp.exp(e)

            @plsc.parallel_loop(0, EC, 1, unroll=4)
            def _(jj):
                pj = plsc.load_gather(pbuf,
                                      [jnp.zeros((16,), jnp.int32) + jj])
                for k in range(DP // 16):
                    rw[jj, pl.ds(k * 16, 16)] = (rw[jj, pl.ds(k * 16, 16)]
                                                 * pj)

            pltpu.sync_copy(rw, acc.at[jd.at[j]], add=True)

        idx_fetch(0, is0, id0, smi0)
        idx_wait(0, is0, id0, smi0)
        gather(is0, id0, 0, rows0, adb0, smr0, sma0)
        gather(is0, id0, 1, rows1, adb1, smr1, sma1)
        idx_fetch(1, is1, id1, smi1)

        def quad(u, carry):
            gwait(is0, id0, 0, rows0, adb0, smr0, sma0)
            compute_scatter(id0, 0, rows0, adb0)
            idx_wait(2 * u + 1, is1, id1, smi1)
            gather(is1, id1, 0, rows0, adb0, smr0, sma0)
            gwait(is0, id0, 1, rows1, adb1, smr1, sma1)
            compute_scatter(id0, 1, rows1, adb1)
            gather(is1, id1, 1, rows1, adb1, smr1, sma1)
            idx_fetch(2 * u + 2, is0, id0, smi0)
            gwait(is1, id1, 0, rows0, adb0, smr0, sma0)
            compute_scatter(id1, 0, rows0, adb0)
            idx_wait(2 * u + 2, is0, id0, smi0)
            gather(is0, id0, 0, rows0, adb0, smr0, sma0)
            gwait(is1, id1, 1, rows1, adb1, smr1, sma1)
            compute_scatter(id1, 1, rows1, adb1)
            gather(is0, id0, 1, rows1, adb1, smr1, sma1)
            idx_fetch(2 * u + 3, is1, id1, smi1)
            return carry

        lax.fori_loop(0, NP // 2, quad, 0)
        gwait(is0, id0, 0, rows0, adb0, smr0, sma0)
        compute_scatter(id0, 0, rows0, adb0)
        gwait(is0, id0, 1, rows1, adb1, smr1, sma1)
        compute_scatter(id0, 1, rows1, adb1)
        idx_wait(NP, is1, id1, smi1)
        plsc.subcore_barrier()

        def bat(c, carry):
            off = sid * BPT + c * GC
            pltpu.sync_copy(bat_hbm.at[pl.ds(off, GC)], bidx)
            pltpu.async_copy(acc.at[bidx], g0, smr0)
            pltpu.async_copy(dst_tab_hbm.at[bidx], g1, smr1)
            pltpu.make_async_copy(acc.at[bidx], g0, smr0).wait()
            pltpu.sync_copy(g0, part_out.at[pl.ds(off, GC), :])
            pltpu.make_async_copy(dst_tab_hbm.at[bidx], g1, smr1).wait()
            pltpu.sync_copy(g1, emb_out.at[pl.ds(off, GC), :])
            return carry

        lax.fori_loop(0, BPT // GC, bat, 0)

    @pl.when(cid == 0)
    def _():
        one_side(adu_hbm, srcu_hbm, dstu_hbm, ip_hbm, up_hbm,
                 u_hbm, pu_out, ur_out)

    @pl.when(cid == 1)
    def _():
        one_side(adi_hbm, srci_hbm, dsti_hbm, up_hbm, ip_hbm,
                 i_hbm, pi_out, ir_out)


def _sc_edges(user_pad, item_pad, adu, adi,
              src_u, dst_u, src_i, dst_i, u_idx, i_idx, zrows):
    mesh = plsc.VectorSubcoreMesh(core_axis_name="c", subcore_axis_name="s")
    f32 = jnp.float32
    kern = functools.partial(
        pl.kernel,
        mesh=mesh,
        compiler_params=pltpu.CompilerParams(needs_layout_passes=False,
                                             use_tc_tiling_on_sc=False),
        out_type=(jax.ShapeDtypeStruct((B, DP), f32),
                  jax.ShapeDtypeStruct((B, DP), f32),
                  jax.ShapeDtypeStruct((B, DP), f32),
                  jax.ShapeDtypeStruct((B, DP), f32)),
        scratch_types=[
            pltpu.VMEM_SHARED((NPAD, DP), f32),
            pltpu.VMEM((2, EC), jnp.int32),
            pltpu.VMEM((2, EC), jnp.int32),
            pltpu.VMEM((EC, DP), f32),
            pltpu.VMEM((EC,), f32),
            pltpu.VMEM((2, EC), jnp.int32),
            pltpu.VMEM((2, EC), jnp.int32),
            pltpu.VMEM((EC, DP), f32),
            pltpu.VMEM((EC,), f32),
            pltpu.VMEM((EC,), f32),
            pltpu.VMEM((GC, DP), f32),
            pltpu.VMEM((GC, DP), f32),
            pltpu.VMEM((GC,), jnp.int32),
            pltpu.SemaphoreType.DMA,
            pltpu.SemaphoreType.DMA,
            pltpu.SemaphoreType.DMA,
            pltpu.SemaphoreType.DMA,
            pltpu.SemaphoreType.DMA,
            pltpu.SemaphoreType.DMA,
        ],
    )(_sc_edges_body)
    return kern(user_pad, item_pad, adu, adi,
                src_u, dst_u, src_i, dst_i, u_idx, i_idx, zrows)


def _dense_body(pu_ref, pi_ref, ur_ref, ir_ref,
                wsu_ref, bsu_ref, wnu_ref, bnu_ref, wfu_ref,
                wsi_ref, bsi_ref, wni_ref, bni_ref, wfi_ref,
                out_ref):
    f32 = jnp.float32
    hp = "default"

    def side(p_ref, e_ref, ws, bs, wn, bn, wf):
        num = p_ref[:, 0:D]
        s = p_ref[:, D:D + 1]
        s = jnp.where(s > 0.0, s, 1.0)
        h = num / s
        nb = jax.nn.relu(jnp.dot(h, wn[...], preferred_element_type=f32,
                                 precision=hp) + bn[...])
        sf = jax.nn.relu(jnp.dot(e_ref[:, 0:D], ws[...],
                                 preferred_element_type=f32,
                                 precision=hp) + bs[...])
        return jax.nn.relu(
            jnp.dot(sf, wf[0:D, :], preferred_element_type=f32, precision=hp)
            + jnp.dot(nb, wf[D:2 * D, :], preferred_element_type=f32,
                      precision=hp))

    out_ref[:, 0:D] = side(pu_ref, ur_ref, wsu_ref, bsu_ref,
                           wnu_ref, bnu_ref, wfu_ref)
    out_ref[:, D:2 * D] = side(pi_ref, ir_ref, wsi_ref, bsi_ref,
                               wni_ref, bni_ref, wfi_ref)


def _dense_stage(part_u, part_i, ue_rows, ie_rows,
                 W_self_u, b_self_u, W_nb_u, b_nb_u, W_fc_u,
                 W_self_i, b_self_i, W_nb_i, b_nb_i, W_fc_i):
    psp = pl.BlockSpec((_DENSE_BLK, DP), lambda b: (b, 0))
    rsp = pl.BlockSpec((_DENSE_BLK, DP), lambda b: (b, 0))
    wsp = lambda r, c: pl.BlockSpec((r, c), lambda b: (0, 0))
    bsp = pl.BlockSpec((D,), lambda b: (0,))
    return pl.pallas_call(
        _dense_body,
        grid=(B // _DENSE_BLK,),
        in_specs=[psp, psp, rsp, rsp,
                  wsp(D, D), bsp, wsp(D, D), bsp, wsp(2 * D, D),
                  wsp(D, D), bsp, wsp(D, D), bsp, wsp(2 * D, D)],
        out_specs=pl.BlockSpec((_DENSE_BLK, 2 * D), lambda b: (b, 0)),
        out_shape=jax.ShapeDtypeStruct((B, 2 * D), jnp.float32),
    )(part_u, part_i, ue_rows, ie_rows,
      W_self_u, b_self_u, W_nb_u, b_nb_u, W_fc_u,
      W_self_i, b_self_i, W_nb_i, b_nb_i, W_fc_i)


def kernel(user_emb, item_emb, W_attn_u, W_attn_i, W_fc_u, W_fc_i,
           W_self_u, b_self_u, W_self_i, b_self_i,
           W_nb_u, b_nb_u, W_nb_i, b_nb_i, edge_iu, edge_ui, u, i):
    Wu2 = jnp.concatenate([W_attn_u[D:], W_attn_i[:D]], axis=1)
    Wi2 = jnp.concatenate([W_attn_u[:D], W_attn_i[D:]], axis=1)
    user_pad, item_pad, AU, AI = _prep(user_emb, item_emb, Wu2, Wi2)
    adu = AU[:, 0]
    adi = AI[:, 0]
    zrows = jnp.zeros((RPT, DP), jnp.float32)
    part_u, part_i, ue_rows, ie_rows = _sc_edges(
        user_pad, item_pad, adu, adi,
        edge_iu[0].reshape(E // EC, EC), edge_iu[1].reshape(E // EC, EC),
        edge_ui[0].reshape(E // EC, EC), edge_ui[1].reshape(E // EC, EC),
        u, i, zrows)
    return _dense_stage(part_u, part_i, ue_rows, ie_rows,
                        W_self_u, b_self_u, W_nb_u, b_nb_u, W_fc_u,
                        W_self_i, b_self_i, W_nb_i, b_nb_i, W_fc_i)
